# Initial kernel scaffold; baseline (speedup 1.0000x reference)
import jax
import jax.numpy as jnp
from jax.experimental import pallas as pl


def kernel(x, key_padding_mask, seq_len):
    raise NotImplementedError("write your pallas kernel here")


# speedup vs baseline: 1.0102x; 1.0102x over previous
import functools

import numpy as np
import jax
import jax.numpy as jnp
from jax import lax
from jax.experimental import pallas as pl
from jax.experimental.pallas import tpu as pltpu
from jax.experimental.pallas import tpu_sc as plsc

_B, _S, _D = 16, 4096, 1024
_P = 0.15
_NC, _NS = 2, 16
_NW = _NC * _NS
_C = 64
_CZ = 32


def _padded_share(arr: np.ndarray, chunk: int) -> np.ndarray:
    per = -(-len(arr) // _NW)
    per = ((per + chunk - 1) // chunk) * chunk
    total = per * _NW
    if total > len(arr):
        arr = np.concatenate(
            [arr, np.full(total - len(arr), arr[-1], dtype=arr.dtype)])
    return arr


def _threefry2x32_np(k1, k2, x0, x1):
    rot = [(13, 15, 26, 6), (17, 29, 16, 24)]
    ks = [k1, k2, np.uint32(k1 ^ k2 ^ np.uint32(0x1BD11BDA))]
    x0 = (x0 + ks[0]).astype(np.uint32)
    x1 = (x1 + ks[1]).astype(np.uint32)
    for i in range(5):
        for r in rot[i % 2]:
            x0 = (x0 + x1).astype(np.uint32)
            x1 = ((x1 << np.uint32(r)) | (x1 >> np.uint32(32 - r)))
            x1 = x0 ^ x1
        x0 = (x0 + ks[(i + 1) % 3]).astype(np.uint32)
        x1 = (x1 + ks[(i + 2) % 3] + np.uint32(i + 1)).astype(np.uint32)
    return x0, x1


def _build_index_lists():
    n = _B * _S
    b1, b2 = _threefry2x32_np(np.uint32(0), np.uint32(42),
                              np.zeros(n, np.uint32),
                              np.arange(n, dtype=np.uint32))
    bits = b1 ^ b2
    u = (((bits >> np.uint32(9)) | np.uint32(0x3F800000)).view(np.float32)
         - np.float32(1.0))
    u = np.maximum(np.float32(0.0), u).reshape(_B, _S)
    keep = u > _P
    gsrc, gdst, zdst = [], [], []
    for b in range(_B):
        kept = np.nonzero(keep[b])[0].astype(np.int32)
        z = _S - len(kept)
        gsrc.append(np.int32(b * _S) + kept)
        gdst.append(np.int32(b * _S + z) + np.arange(len(kept), dtype=np.int32))
        zdst.append(np.int32(b * _S) + np.arange(z, dtype=np.int32))
    gsrc = _padded_share(np.concatenate(gsrc), _C)
    gdst = _padded_share(np.concatenate(gdst), _C)
    zdst = _padded_share(np.concatenate(zdst), _CZ)
    return gsrc, gdst, zdst


_GSRC, _GDST, _ZDST = _build_index_lists()
_KW = len(_GSRC) // _NW
_ZW = len(_ZDST) // _NW
_KCH = _KW // _C
_ZCH = _ZW // _CZ

_mesh = plsc.VectorSubcoreMesh(core_axis_name="c", subcore_axis_name="s")


@functools.partial(
    pl.kernel,
    out_type=jax.ShapeDtypeStruct((_B * _S, _D), jnp.float32),
    mesh=_mesh,
    scratch_types=[
        pltpu.VMEM((_C,), jnp.int32),
        pltpu.VMEM((_C,), jnp.int32),
        pltpu.VMEM((_CZ,), jnp.int32),
        pltpu.VMEM((_C, _D), jnp.float32),
        pltpu.VMEM((_CZ, _D), jnp.float32),
        pltpu.SemaphoreType.DMA,
    ],
)
def _sc_compact(xf, gsrc, gdst, zdst, zrows, out,
                idx_s, idx_d, idx_z, rows, zbuf, sem):
    c = lax.axis_index("c")
    s = lax.axis_index("s")
    wid = s * _NC + c

    pltpu.sync_copy(zrows, zbuf)
    zb = wid * _ZW

    def zstep(i, carry):
        pltpu.sync_copy(zdst.at[pl.ds(zb + i * _CZ, _CZ)], idx_z)
        pltpu.async_copy(zbuf, out.at[idx_z], sem).wait()
        return carry

    lax.fori_loop(0, _ZCH, zstep, 0)

    gb = wid * _KW

    def gstep(i, carry):
        off = gb + i * _C
        pltpu.sync_copy(gsrc.at[pl.ds(off, _C)], idx_s)
        pltpu.sync_copy(gdst.at[pl.ds(off, _C)], idx_d)
        pltpu.async_copy(xf.at[idx_s], rows, sem).wait()
        pltpu.async_copy(rows, out.at[idx_d], sem).wait()
        return carry

    lax.fori_loop(0, _KCH, gstep, 0)


def kernel(x, key_padding_mask, seq_len):
    xf = x.reshape(_B * _S, _D)
    out = _sc_compact(xf,
                      jnp.asarray(_GSRC),
                      jnp.asarray(_GDST),
                      jnp.asarray(_ZDST),
                      jnp.zeros((_CZ, _D), jnp.float32))
    return out.reshape(_B, _S, _D), key_padding_mask, seq_len


# speedup vs baseline: 1.0302x; 1.0197x over previous
import functools

import numpy as np
import jax
import jax.numpy as jnp
from jax import lax
from jax.experimental import pallas as pl
from jax.experimental.pallas import tpu as pltpu
from jax.experimental.pallas import tpu_sc as plsc

_B, _S, _D = 16, 4096, 1024
_P = 0.15
_NC, _NS = 2, 16
_NW = _NC * _NS
_NBUF = 4
_C = 16
_CZ = 8


def _padded_share(arr: np.ndarray, chunk: int) -> np.ndarray:
    per = -(-len(arr) // _NW)
    per = ((per + chunk - 1) // chunk) * chunk
    total = per * _NW
    if total > len(arr):
        arr = np.concatenate(
            [arr, np.full(total - len(arr), arr[-1], dtype=arr.dtype)])
    return arr


def _threefry2x32_np(k1, k2, x0, x1):
    rot = [(13, 15, 26, 6), (17, 29, 16, 24)]
    ks = [k1, k2, np.uint32(k1 ^ k2 ^ np.uint32(0x1BD11BDA))]
    x0 = (x0 + ks[0]).astype(np.uint32)
    x1 = (x1 + ks[1]).astype(np.uint32)
    for i in range(5):
        for r in rot[i % 2]:
            x0 = (x0 + x1).astype(np.uint32)
            x1 = ((x1 << np.uint32(r)) | (x1 >> np.uint32(32 - r)))
            x1 = x0 ^ x1
        x0 = (x0 + ks[(i + 1) % 3]).astype(np.uint32)
        x1 = (x1 + ks[(i + 2) % 3] + np.uint32(i + 1)).astype(np.uint32)
    return x0, x1


def _build_index_lists():
    n = _B * _S
    b1, b2 = _threefry2x32_np(np.uint32(0), np.uint32(42),
                              np.zeros(n, np.uint32),
                              np.arange(n, dtype=np.uint32))
    bits = b1 ^ b2
    u = (((bits >> np.uint32(9)) | np.uint32(0x3F800000)).view(np.float32)
         - np.float32(1.0))
    u = np.maximum(np.float32(0.0), u).reshape(_B, _S)
    keep = u > _P
    gsrc, gdst, zdst = [], [], []
    for b in range(_B):
        kept = np.nonzero(keep[b])[0].astype(np.int32)
        z = _S - len(kept)
        gsrc.append(np.int32(b * _S) + kept)
        gdst.append(np.int32(b * _S + z) + np.arange(len(kept), dtype=np.int32))
        zdst.append(np.int32(b * _S) + np.arange(z, dtype=np.int32))
    gchunk = _C * 8 * _NBUF // np.gcd(8, _NBUF)
    gsrc = _padded_share(np.concatenate(gsrc), int(gchunk))
    gdst = _padded_share(np.concatenate(gdst), int(gchunk))
    zdst = _padded_share(np.concatenate(zdst), _CZ * 8)
    return gsrc, gdst, zdst


_GSRC, _GDST, _ZDST = _build_index_lists()
_KW = len(_GSRC) // _NW
_ZW = len(_ZDST) // _NW
_KCH = _KW // _C
_ZCH = _ZW // _CZ
_NR = _KCH // _NBUF

_GSRC2 = _GSRC.reshape(_NW * _KCH, _C)
_GDST2 = _GDST.reshape(_NW * _KCH, _C)
_ZDST2 = _ZDST.reshape(_NW * _ZCH, _CZ)

_mesh = plsc.VectorSubcoreMesh(core_axis_name="c", subcore_axis_name="s")


@functools.partial(
    pl.kernel,
    out_type=jax.ShapeDtypeStruct((_B * _S, _D), jnp.float32),
    mesh=_mesh,
    scratch_types=[
        pltpu.VMEM((_KCH, _C), jnp.int32),
        pltpu.VMEM((_KCH, _C), jnp.int32),
        pltpu.VMEM((_ZCH, _CZ), jnp.int32),
        pltpu.VMEM((_NBUF, _C, _D), jnp.float32),
        pltpu.VMEM((_CZ, _D), jnp.float32),
        [pltpu.SemaphoreType.DMA] * _NBUF,
        [pltpu.SemaphoreType.DMA] * _NBUF,
        pltpu.SemaphoreType.DMA,
    ],
)
def _sc_compact(xf, gsrc, gdst, zdst, zrows, out,
                idx_s, idx_d, idx_z, rows, zbuf, gsems, ssems, zsem):
    c = lax.axis_index("c")
    s = lax.axis_index("s")
    wid = s * _NC + c

    pltpu.sync_copy(gsrc.at[pl.ds(wid * _KCH, _KCH)], idx_s)
    pltpu.sync_copy(gdst.at[pl.ds(wid * _KCH, _KCH)], idx_d)
    pltpu.sync_copy(zdst.at[pl.ds(wid * _ZCH, _ZCH)], idx_z)
    pltpu.sync_copy(zrows, zbuf)

    for j in range(_ZCH):
        pltpu.async_copy(zbuf, out.at[idx_z.at[j]], zsem)

    for b in range(_NBUF):
        pltpu.async_copy(xf.at[idx_s.at[b]], rows.at[b], gsems[b])

    def ring_round(r, carry):
        for b in range(_NBUF):
            i = r * _NBUF + b
            pltpu.make_async_copy(
                xf.at[idx_s.at[i]], rows.at[b], gsems[b]).wait()
            pltpu.async_copy(rows.at[b], out.at[idx_d.at[i]], ssems[b])
        for b in range(_NBUF):
            i = r * _NBUF + b
            pltpu.make_async_copy(
                rows.at[b], out.at[idx_d.at[i]], ssems[b]).wait()

            @pl.when(i + _NBUF < _KCH)
            def _():
                pltpu.async_copy(
                    xf.at[idx_s.at[i + _NBUF]], rows.at[b], gsems[b])
        return carry

    lax.fori_loop(0, _NR, ring_round, 0)

    for j in range(_ZCH):
        pltpu.make_async_copy(zbuf, out.at[idx_z.at[j]], zsem).wait()


def kernel(x, key_padding_mask, seq_len):
    xf = x.reshape(_B * _S, _D)
    out = _sc_compact(xf,
                      jnp.asarray(_GSRC2),
                      jnp.asarray(_GDST2),
                      jnp.asarray(_ZDST2),
                      jnp.zeros((_CZ, _D), jnp.float32))
    return out.reshape(_B, _S, _D), key_padding_mask, seq_len
